# baseline (speedup 1.0000x reference)
import jax
import jax.numpy as jnp
from jax import lax
from jax.experimental import pallas as pl
from jax.experimental.pallas import tpu as pltpu
from jax.experimental.pallas import tpu_sc as plsc

F32 = jnp.float32
BF16 = jnp.bfloat16
I32 = jnp.int32

T = 2048
D = 768
E = 16
I = 512
G = 4
TOPK_G = 2
K = 2
BT = 128
NB = (T * K) // BT + E
RS = 1.0

NC, NS = 2, 16
NW = NC * NS
TPW = T // NW


def _sigmoid(x):
    return 1.0 / (1.0 + jnp.exp(-x))


def _silu(x):
    return x * _sigmoid(x)


def _gate_body(x_ref, gwt_ref, gb_ref, pos0_ref, pos1_ref, w0_ref, w1_ref,
               be_ref, nb_ref):
    x = x_ref[...]
    logits = jnp.dot(x, gwt_ref[...], preferred_element_type=F32)
    st = _sigmoid(jnp.transpose(logits))
    s = st + gb_ref[...]

    erow = lax.broadcasted_iota(I32, (E, T), 0)

    gsc = []
    for g in range(G):
        rows = [s[g * 4 + j:g * 4 + j + 1, :] for j in range(4)]
        best = rows[0] + rows[1]
        for a in range(4):
            for b in range(a + 1, 4):
                if (a, b) != (0, 1):
                    best = jnp.maximum(best, rows[a] + rows[b])
        gsc.append(best)

    keep16 = jnp.zeros((E, T), dtype=jnp.bool_)
    for g in range(G):
        rank = jnp.zeros((1, T), dtype=I32)
        for g2 in range(G):
            if g2 == g:
                continue
            beats = (gsc[g2] > gsc[g]) | ((gsc[g2] == gsc[g]) & (g2 < g))
            rank = rank + beats.astype(I32)
        keep16 = keep16 | (((erow // 4) == g) & (rank < TOPK_G))

    sm = jnp.where(keep16, s, -1e30)

    r = jnp.zeros((E, T), dtype=I32)
    for e2 in range(E):
        row = sm[e2:e2 + 1, :]
        beats = (row > sm) | ((row == sm) & (e2 < erow))
        r = r + beats.astype(I32)
    sel = r < K

    wsel = jnp.where(sel, st, 0.0)
    denom = jnp.sum(wsel, axis=0, keepdims=True) + 1e-6
    cw = wsel / denom * RS

    c = sel.astype(F32)
    sh = 1
    while sh < T:
        c = c + jnp.concatenate(
            [jnp.zeros((E, sh), F32), c[:, :T - sh]], axis=1)
        sh *= 2
    rank_t = c - sel.astype(F32)
    counts = c[:, T - 1:T]

    pc = jnp.ceil(counts / BT) * BT
    p = pc
    sh = 1
    while sh < E:
        p = p + jnp.concatenate([jnp.zeros((sh, 1), F32), p[:E - sh, :]],
                                axis=0)
        sh *= 2
    po = p - pc

    pos16 = po + rank_t

    is0 = sel & (r == 0)
    is1 = sel & (r == 1)
    pos0 = jnp.sum(jnp.where(is0, pos16, 0.0), axis=0, keepdims=True)
    pos1 = jnp.sum(jnp.where(is1, pos16, 0.0), axis=0, keepdims=True)
    w0 = jnp.sum(jnp.where(is0, cw, 0.0), axis=0, keepdims=True)
    w1 = jnp.sum(jnp.where(is1, cw, 0.0), axis=0, keepdims=True)
    pos0_ref[...] = jnp.transpose(pos0).astype(I32)
    pos1_ref[...] = jnp.transpose(pos1).astype(I32)
    w0_ref[...] = jnp.transpose(w0)
    w1_ref[...] = jnp.transpose(w1)

    bi = (lax.broadcasted_iota(I32, (NB, E), 0) * BT).astype(F32)
    cmp = (jnp.broadcast_to(jnp.transpose(po), (NB, E)) <= bi).astype(I32)
    be = jnp.sum(cmp, axis=1, keepdims=True) - 1
    be_ref[...] = jnp.clip(be, 0, E - 1)
    total_pad = jnp.sum(pc, axis=0, keepdims=True)
    nb_ref[...] = (total_pad / BT).astype(I32)


def _gating(xf, gate_weight, gate_bias):
    out_shapes = (
        jax.ShapeDtypeStruct((T, 1), I32),
        jax.ShapeDtypeStruct((T, 1), I32),
        jax.ShapeDtypeStruct((T, 1), F32),
        jax.ShapeDtypeStruct((T, 1), F32),
        jax.ShapeDtypeStruct((NB, 1), I32),
        jax.ShapeDtypeStruct((1, 1), I32),
    )
    return pl.pallas_call(
        _gate_body,
        out_shape=out_shapes,
    )(xf, gate_weight.T, gate_bias.reshape(E, 1))


def _sc_scatter_body(xf_hbm, pos0_hbm, pos1_hbm, xg_hbm, idx0_v, idx1_v,
                     rows_v, sem):
    wid = lax.axis_index("s") * NC + lax.axis_index("c")
    base = wid * TPW
    pltpu.sync_copy(pos0_hbm.at[pl.ds(base, TPW)], idx0_v)
    pltpu.sync_copy(pos1_hbm.at[pl.ds(base, TPW)], idx1_v)
    pltpu.sync_copy(xf_hbm.at[pl.ds(base, TPW)], rows_v)
    cp0 = pltpu.async_copy(rows_v, xg_hbm.at[idx0_v], sem)
    cp1 = pltpu.async_copy(rows_v, xg_hbm.at[idx1_v], sem)
    cp0.wait()
    cp1.wait()


def _sc_scatter(xf, pos0, pos1):
    mesh = plsc.VectorSubcoreMesh(core_axis_name="c", subcore_axis_name="s",
                                  num_cores=NC, num_subcores=NS)
    return pl.kernel(
        _sc_scatter_body,
        out_type=jax.ShapeDtypeStruct((NB * BT, D), F32),
        mesh=mesh,
        scratch_types=[
            pltpu.VMEM((TPW,), I32),
            pltpu.VMEM((TPW,), I32),
            pltpu.VMEM((TPW, D), F32),
            pltpu.SemaphoreType.DMA,
        ],
    )(xf, pos0, pos1)


def _ffn_body(be_ref, nb_ref, xg_ref, w1_ref, w3_ref, w2_ref, out_ref,
              w1b_ref, w3b_ref, w2b_ref):
    b = pl.program_id(0)

    first = b == 0
    changed = jnp.logical_or(first, be_ref[b] != be_ref[jnp.maximum(b - 1, 0)])

    @pl.when(changed)
    def _():
        w1b_ref[...] = w1_ref[0].astype(BF16)
        w3b_ref[...] = w3_ref[0].astype(BF16)
        w2b_ref[...] = w2_ref[0].astype(BF16)

    @pl.when(b < nb_ref[0])
    def _():
        x = xg_ref[...].astype(BF16)
        h1 = lax.dot_general(x, w1b_ref[...], (((1,), (1,)), ((), ())),
                             preferred_element_type=F32)
        h3 = lax.dot_general(x, w3b_ref[...], (((1,), (1,)), ((), ())),
                             preferred_element_type=F32)
        g = (_silu(h1) * h3).astype(BF16)
        out_ref[...] = lax.dot_general(g, w2b_ref[...],
                                       (((1,), (1,)), ((), ())),
                                       preferred_element_type=F32)


def _grouped_ffn(xg, be, nb, W1, W2, W3):
    grid_spec = pltpu.PrefetchScalarGridSpec(
        num_scalar_prefetch=2,
        grid=(NB,),
        in_specs=[
            pl.BlockSpec((BT, D), lambda b, be, nb: (b, 0)),
            pl.BlockSpec((1, I, D), lambda b, be, nb: (be[b], 0, 0)),
            pl.BlockSpec((1, I, D), lambda b, be, nb: (be[b], 0, 0)),
            pl.BlockSpec((1, D, I), lambda b, be, nb: (be[b], 0, 0)),
        ],
        out_specs=pl.BlockSpec((BT, D), lambda b, be, nb: (b, 0)),
        scratch_shapes=[
            pltpu.VMEM((I, D), BF16),
            pltpu.VMEM((I, D), BF16),
            pltpu.VMEM((D, I), BF16),
        ],
    )
    return pl.pallas_call(
        _ffn_body,
        grid_spec=grid_spec,
        out_shape=jax.ShapeDtypeStruct((NB * BT, D), F32),
    )(be, nb, xg, W1, W3, W2)


def _sc_gather_body(yg_hbm, pos0_hbm, pos1_hbm, ya_hbm, yb_hbm, idx0_v,
                    idx1_v, rows0_v, rows1_v, sem0, sem1):
    wid = lax.axis_index("s") * NC + lax.axis_index("c")
    base = wid * TPW
    pltpu.sync_copy(pos0_hbm.at[pl.ds(base, TPW)], idx0_v)
    pltpu.sync_copy(pos1_hbm.at[pl.ds(base, TPW)], idx1_v)
    cp0 = pltpu.async_copy(yg_hbm.at[idx0_v], rows0_v, sem0)
    cp1 = pltpu.async_copy(yg_hbm.at[idx1_v], rows1_v, sem1)
    cp0.wait()
    pltpu.sync_copy(rows0_v, ya_hbm.at[pl.ds(base, TPW)])
    cp1.wait()
    pltpu.sync_copy(rows1_v, yb_hbm.at[pl.ds(base, TPW)])


def _sc_gather(yg, pos0, pos1):
    mesh = plsc.VectorSubcoreMesh(core_axis_name="c", subcore_axis_name="s",
                                  num_cores=NC, num_subcores=NS)
    return pl.kernel(
        _sc_gather_body,
        out_type=(jax.ShapeDtypeStruct((T, D), F32),
                  jax.ShapeDtypeStruct((T, D), F32)),
        mesh=mesh,
        scratch_types=[
            pltpu.VMEM((TPW,), I32),
            pltpu.VMEM((TPW,), I32),
            pltpu.VMEM((TPW, D), F32),
            pltpu.VMEM((TPW, D), F32),
            pltpu.SemaphoreType.DMA,
            pltpu.SemaphoreType.DMA,
        ],
    )(yg, pos0, pos1)


def _shared_body(x_ref, sw1_ref, sw3_ref, sw2_ref, ya_ref, yb_ref, w0_ref,
                 w1_ref, out_ref, sw1b_ref, sw3b_ref, sw2b_ref):
    @pl.when(pl.program_id(0) == 0)
    def _():
        sw1b_ref[...] = sw1_ref[...].astype(BF16)
        sw3b_ref[...] = sw3_ref[...].astype(BF16)
        sw2b_ref[...] = sw2_ref[...].astype(BF16)

    x = x_ref[...].astype(BF16)
    h1 = lax.dot_general(x, sw1b_ref[...], (((1,), (1,)), ((), ())),
                         preferred_element_type=F32)
    h3 = lax.dot_general(x, sw3b_ref[...], (((1,), (1,)), ((), ())),
                         preferred_element_type=F32)
    g = (_silu(h1) * h3).astype(BF16)
    z = lax.dot_general(g, sw2b_ref[...], (((1,), (1,)), ((), ())),
                        preferred_element_type=F32)
    out_ref[...] = (z + w0_ref[...] * ya_ref[...]
                    + w1_ref[...] * yb_ref[...])


def _shared_combine(xf, sw1, sw2, sw3, ya, yb, w0, w1):
    BS = 256
    nblk = T // BS
    return pl.pallas_call(
        _shared_body,
        grid=(nblk,),
        in_specs=[
            pl.BlockSpec((BS, D), lambda b: (b, 0)),
            pl.BlockSpec(sw1.shape, lambda b: (0, 0)),
            pl.BlockSpec(sw3.shape, lambda b: (0, 0)),
            pl.BlockSpec(sw2.shape, lambda b: (0, 0)),
            pl.BlockSpec((BS, D), lambda b: (b, 0)),
            pl.BlockSpec((BS, D), lambda b: (b, 0)),
            pl.BlockSpec((BS, 1), lambda b: (b, 0)),
            pl.BlockSpec((BS, 1), lambda b: (b, 0)),
        ],
        out_specs=pl.BlockSpec((BS, D), lambda b: (b, 0)),
        out_shape=jax.ShapeDtypeStruct((T, D), F32),
        scratch_shapes=[
            pltpu.VMEM(sw1.shape, BF16),
            pltpu.VMEM(sw3.shape, BF16),
            pltpu.VMEM(sw2.shape, BF16),
        ],
    )(xf, sw1, sw3, sw2, ya, yb, w0, w1)


@jax.jit
def kernel(x, gate_weight, gate_bias, W1, W2, W3, sw1, sw2, sw3):
    Bb, Ss, Dd = x.shape
    xf = x.reshape(T, D)

    pos0, pos1, w0, w1, be, nb = _gating(xf, gate_weight, gate_bias)
    pos0v = pos0.reshape(T)
    pos1v = pos1.reshape(T)
    bev = be.reshape(NB)
    nbv = nb.reshape(1)

    xg = _sc_scatter(xf, pos0v, pos1v)
    yg = _grouped_ffn(xg, bev, nbv, W1, W2, W3)
    ya, yb = _sc_gather(yg, pos0v, pos1v)
    out = _shared_combine(xf, sw1, sw2, sw3, ya, yb, w0, w1)
    return out.reshape(Bb, Ss, Dd)


# speedup vs baseline: 1.0263x; 1.0263x over previous
import jax
import jax.numpy as jnp
from jax import lax
from jax.experimental import pallas as pl
from jax.experimental.pallas import tpu as pltpu
from jax.experimental.pallas import tpu_sc as plsc

F32 = jnp.float32
BF16 = jnp.bfloat16
I32 = jnp.int32

T = 2048
D = 768
E = 16
I = 512
G = 4
TOPK_G = 2
K = 2
BT = 128
NB = (T * K) // BT + E
RS = 1.0

NC, NS = 2, 16
NW = NC * NS
TPW = T // NW


def _sigmoid(x):
    return 1.0 / (1.0 + jnp.exp(-x))


def _silu(x):
    return x * _sigmoid(x)


def _gate_body(x_ref, gwt_ref, gb_ref, pos0_ref, pos1_ref, w0_ref, w1_ref,
               be_ref, nb_ref):
    x = x_ref[...]
    logits = jnp.dot(x, gwt_ref[...], preferred_element_type=F32)
    st = _sigmoid(jnp.transpose(logits))
    s = st + gb_ref[...]

    erow = lax.broadcasted_iota(I32, (E, T), 0)

    gsc = []
    for g in range(G):
        rows = [s[g * 4 + j:g * 4 + j + 1, :] for j in range(4)]
        best = rows[0] + rows[1]
        for a in range(4):
            for b in range(a + 1, 4):
                if (a, b) != (0, 1):
                    best = jnp.maximum(best, rows[a] + rows[b])
        gsc.append(best)

    keep16 = jnp.zeros((E, T), dtype=jnp.bool_)
    for g in range(G):
        rank = jnp.zeros((1, T), dtype=I32)
        for g2 in range(G):
            if g2 == g:
                continue
            beats = (gsc[g2] > gsc[g]) | ((gsc[g2] == gsc[g]) & (g2 < g))
            rank = rank + beats.astype(I32)
        keep16 = keep16 | (((erow // 4) == g) & (rank < TOPK_G))

    sm = jnp.where(keep16, s, -1e30)

    r = jnp.zeros((E, T), dtype=I32)
    for e2 in range(E):
        row = sm[e2:e2 + 1, :]
        beats = (row > sm) | ((row == sm) & (e2 < erow))
        r = r + beats.astype(I32)
    sel = r < K

    wsel = jnp.where(sel, st, 0.0)
    denom = jnp.sum(wsel, axis=0, keepdims=True) + 1e-6
    cw = wsel / denom * RS

    c = sel.astype(F32)
    sh = 1
    while sh < T:
        c = c + jnp.concatenate(
            [jnp.zeros((E, sh), F32), c[:, :T - sh]], axis=1)
        sh *= 2
    rank_t = c - sel.astype(F32)
    counts = c[:, T - 1:T]

    pc = jnp.ceil(counts / BT) * BT
    p = pc
    sh = 1
    while sh < E:
        p = p + jnp.concatenate([jnp.zeros((sh, 1), F32), p[:E - sh, :]],
                                axis=0)
        sh *= 2
    po = p - pc

    pos16 = po + rank_t

    is0 = sel & (r == 0)
    is1 = sel & (r == 1)
    pos0 = jnp.sum(jnp.where(is0, pos16, 0.0), axis=0, keepdims=True)
    pos1 = jnp.sum(jnp.where(is1, pos16, 0.0), axis=0, keepdims=True)
    w0 = jnp.sum(jnp.where(is0, cw, 0.0), axis=0, keepdims=True)
    w1 = jnp.sum(jnp.where(is1, cw, 0.0), axis=0, keepdims=True)
    pos0_ref[...] = jnp.transpose(pos0).astype(I32)
    pos1_ref[...] = jnp.transpose(pos1).astype(I32)
    w0_ref[...] = jnp.transpose(w0)
    w1_ref[...] = jnp.transpose(w1)

    bi = (lax.broadcasted_iota(I32, (NB, E), 0) * BT).astype(F32)
    cmp = (jnp.broadcast_to(jnp.transpose(po), (NB, E)) <= bi).astype(I32)
    be = jnp.sum(cmp, axis=1, keepdims=True) - 1
    be_ref[...] = jnp.clip(be, 0, E - 1)
    total_pad = jnp.sum(pc, axis=0, keepdims=True)
    nb_ref[...] = (total_pad / BT).astype(I32)


def _gating(xf, gate_weight, gate_bias):
    out_shapes = (
        jax.ShapeDtypeStruct((T, 1), I32),
        jax.ShapeDtypeStruct((T, 1), I32),
        jax.ShapeDtypeStruct((T, 1), F32),
        jax.ShapeDtypeStruct((T, 1), F32),
        jax.ShapeDtypeStruct((NB, 1), I32),
        jax.ShapeDtypeStruct((1, 1), I32),
    )
    return pl.pallas_call(
        _gate_body,
        out_shape=out_shapes,
    )(xf, gate_weight.T, gate_bias.reshape(E, 1))


def _sc_scatter_body(xf_hbm, pos0_hbm, pos1_hbm, xg_hbm, idx0_v, idx1_v,
                     rows_v, sem):
    wid = lax.axis_index("s") * NC + lax.axis_index("c")
    base = wid * TPW
    pltpu.sync_copy(pos0_hbm.at[pl.ds(base, TPW)], idx0_v)
    pltpu.sync_copy(pos1_hbm.at[pl.ds(base, TPW)], idx1_v)
    pltpu.sync_copy(xf_hbm.at[pl.ds(base, TPW)], rows_v)
    cp0 = pltpu.async_copy(rows_v, xg_hbm.at[idx0_v], sem)
    cp1 = pltpu.async_copy(rows_v, xg_hbm.at[idx1_v], sem)
    cp0.wait()
    cp1.wait()


def _sc_scatter(xf, pos0, pos1):
    mesh = plsc.VectorSubcoreMesh(core_axis_name="c", subcore_axis_name="s",
                                  num_cores=NC, num_subcores=NS)
    return pl.kernel(
        _sc_scatter_body,
        out_type=jax.ShapeDtypeStruct((NB * BT, D), F32),
        mesh=mesh,
        scratch_types=[
            pltpu.VMEM((TPW,), I32),
            pltpu.VMEM((TPW,), I32),
            pltpu.VMEM((TPW, D), F32),
            pltpu.SemaphoreType.DMA,
        ],
    )(xf, pos0, pos1)


def _w_copies(w1_ref, w3_ref, w2_ref, e, s, w1s, w3s, w2s, sem1, sem3, sem2):
    return (
        pltpu.make_async_copy(w1_ref.at[e], w1s.at[s], sem1.at[s]),
        pltpu.make_async_copy(w3_ref.at[e], w3s.at[s], sem3.at[s]),
        pltpu.make_async_copy(w2_ref.at[e], w2s.at[s], sem2.at[s]),
    )


def _ffn_body(be_ref, nb_ref, xg_ref, w1_ref, w3_ref, w2_ref, out_ref,
              w1s, w3s, w2s, cnt_ref, sem1, sem3, sem2):
    b = pl.program_id(0)
    prev = be_ref[jnp.maximum(b - 1, 0)]
    changed = jnp.logical_or(b == 0, be_ref[b] != prev)

    @pl.when(b == 0)
    def _():
        cnt_ref[0] = 0
        for cp in _w_copies(w1_ref, w3_ref, w2_ref, be_ref[0], 0,
                            w1s, w3s, w2s, sem1, sem3, sem2):
            cp.start()

    @pl.when(jnp.logical_and(b > 0, changed))
    def _():
        cnt_ref[0] = cnt_ref[0] + 1

    slot = lax.rem(cnt_ref[0], 2)

    @pl.when(changed)
    def _():
        for cp in _w_copies(w1_ref, w3_ref, w2_ref, be_ref[b], slot,
                            w1s, w3s, w2s, sem1, sem3, sem2):
            cp.wait()

    nxt = be_ref[jnp.minimum(b + 1, NB - 1)]
    do_pf = jnp.logical_and(b + 1 < NB, nxt != be_ref[b])

    @pl.when(do_pf)
    def _():
        for cp in _w_copies(w1_ref, w3_ref, w2_ref, nxt, 1 - slot,
                            w1s, w3s, w2s, sem1, sem3, sem2):
            cp.start()

    @pl.when(b < nb_ref[0])
    def _():
        x = xg_ref[...]
        h1 = lax.dot_general(x, w1s[slot], (((1,), (1,)), ((), ())),
                             preferred_element_type=F32)
        h3 = lax.dot_general(x, w3s[slot], (((1,), (1,)), ((), ())),
                             preferred_element_type=F32)
        g = _silu(h1) * h3
        out_ref[...] = lax.dot_general(g, w2s[slot],
                                       (((1,), (1,)), ((), ())),
                                       preferred_element_type=F32)


def _grouped_ffn(xg, be, nb, W1, W2, W3):
    grid_spec = pltpu.PrefetchScalarGridSpec(
        num_scalar_prefetch=2,
        grid=(NB,),
        in_specs=[
            pl.BlockSpec((BT, D), lambda b, be, nb: (b, 0)),
            pl.BlockSpec(memory_space=pl.ANY),
            pl.BlockSpec(memory_space=pl.ANY),
            pl.BlockSpec(memory_space=pl.ANY),
        ],
        out_specs=pl.BlockSpec((BT, D), lambda b, be, nb: (b, 0)),
        scratch_shapes=[
            pltpu.VMEM((2, I, D), F32),
            pltpu.VMEM((2, I, D), F32),
            pltpu.VMEM((2, D, I), F32),
            pltpu.SMEM((1,), I32),
            pltpu.SemaphoreType.DMA((2,)),
            pltpu.SemaphoreType.DMA((2,)),
            pltpu.SemaphoreType.DMA((2,)),
        ],
    )
    return pl.pallas_call(
        _ffn_body,
        grid_spec=grid_spec,
        out_shape=jax.ShapeDtypeStruct((NB * BT, D), F32),
    )(be, nb, xg, W1, W3, W2)


def _sc_gather_body(yg_hbm, pos0_hbm, pos1_hbm, ya_hbm, yb_hbm, idx0_v,
                    idx1_v, rows0_v, rows1_v, sem0, sem1):
    wid = lax.axis_index("s") * NC + lax.axis_index("c")
    base = wid * TPW
    pltpu.sync_copy(pos0_hbm.at[pl.ds(base, TPW)], idx0_v)
    pltpu.sync_copy(pos1_hbm.at[pl.ds(base, TPW)], idx1_v)
    cp0 = pltpu.async_copy(yg_hbm.at[idx0_v], rows0_v, sem0)
    cp1 = pltpu.async_copy(yg_hbm.at[idx1_v], rows1_v, sem1)
    cp0.wait()
    pltpu.sync_copy(rows0_v, ya_hbm.at[pl.ds(base, TPW)])
    cp1.wait()
    pltpu.sync_copy(rows1_v, yb_hbm.at[pl.ds(base, TPW)])


def _sc_gather(yg, pos0, pos1):
    mesh = plsc.VectorSubcoreMesh(core_axis_name="c", subcore_axis_name="s",
                                  num_cores=NC, num_subcores=NS)
    return pl.kernel(
        _sc_gather_body,
        out_type=(jax.ShapeDtypeStruct((T, D), F32),
                  jax.ShapeDtypeStruct((T, D), F32)),
        mesh=mesh,
        scratch_types=[
            pltpu.VMEM((TPW,), I32),
            pltpu.VMEM((TPW,), I32),
            pltpu.VMEM((TPW, D), F32),
            pltpu.VMEM((TPW, D), F32),
            pltpu.SemaphoreType.DMA,
            pltpu.SemaphoreType.DMA,
        ],
    )(yg, pos0, pos1)


def _shared_body(x_ref, sw1_ref, sw3_ref, sw2_ref, ya_ref, yb_ref, w0_ref,
                 w1_ref, out_ref, sw1b_ref, sw3b_ref, sw2b_ref):
    @pl.when(pl.program_id(0) == 0)
    def _():
        sw1b_ref[...] = sw1_ref[...].astype(BF16)
        sw3b_ref[...] = sw3_ref[...].astype(BF16)
        sw2b_ref[...] = sw2_ref[...].astype(BF16)

    x = x_ref[...].astype(BF16)
    h1 = lax.dot_general(x, sw1b_ref[...], (((1,), (1,)), ((), ())),
                         preferred_element_type=F32)
    h3 = lax.dot_general(x, sw3b_ref[...], (((1,), (1,)), ((), ())),
                         preferred_element_type=F32)
    g = (_silu(h1) * h3).astype(BF16)
    z = lax.dot_general(g, sw2b_ref[...], (((1,), (1,)), ((), ())),
                        preferred_element_type=F32)
    out_ref[...] = (z + w0_ref[...] * ya_ref[...]
                    + w1_ref[...] * yb_ref[...])


def _shared_combine(xf, sw1, sw2, sw3, ya, yb, w0, w1):
    BS = 256
    nblk = T // BS
    return pl.pallas_call(
        _shared_body,
        grid=(nblk,),
        in_specs=[
            pl.BlockSpec((BS, D), lambda b: (b, 0)),
            pl.BlockSpec(sw1.shape, lambda b: (0, 0)),
            pl.BlockSpec(sw3.shape, lambda b: (0, 0)),
            pl.BlockSpec(sw2.shape, lambda b: (0, 0)),
            pl.BlockSpec((BS, D), lambda b: (b, 0)),
            pl.BlockSpec((BS, D), lambda b: (b, 0)),
            pl.BlockSpec((BS, 1), lambda b: (b, 0)),
            pl.BlockSpec((BS, 1), lambda b: (b, 0)),
        ],
        out_specs=pl.BlockSpec((BS, D), lambda b: (b, 0)),
        out_shape=jax.ShapeDtypeStruct((T, D), F32),
        scratch_shapes=[
            pltpu.VMEM(sw1.shape, BF16),
            pltpu.VMEM(sw3.shape, BF16),
            pltpu.VMEM(sw2.shape, BF16),
        ],
    )(xf, sw1, sw3, sw2, ya, yb, w0, w1)


@jax.jit
def kernel(x, gate_weight, gate_bias, W1, W2, W3, sw1, sw2, sw3):
    Bb, Ss, Dd = x.shape
    xf = x.reshape(T, D)

    pos0, pos1, w0, w1, be, nb = _gating(xf, gate_weight, gate_bias)
    pos0v = pos0.reshape(T)
    pos1v = pos1.reshape(T)
    bev = be.reshape(NB)
    nbv = nb.reshape(1)

    xg = _sc_scatter(xf, pos0v, pos1v)
    yg = _grouped_ffn(xg, bev, nbv, W1, W2, W3)
    ya, yb = _sc_gather(yg, pos0v, pos1v)
    out = _shared_combine(xf, sw1, sw2, sw3, ya, yb, w0, w1)
    return out.reshape(Bb, Ss, Dd)


# speedup vs baseline: 1.1069x; 1.0786x over previous
import jax
import jax.numpy as jnp
from jax import lax
from jax.experimental import pallas as pl
from jax.experimental.pallas import tpu as pltpu
from jax.experimental.pallas import tpu_sc as plsc

F32 = jnp.float32
BF16 = jnp.bfloat16
I32 = jnp.int32

T = 2048
D = 768
E = 16
I = 512
G = 4
TOPK_G = 2
K = 2
BT = 256
NB = (T * K) // BT + E
RS = 1.0

NC, NS = 2, 16
NW = NC * NS
TPW = T // NW


def _sigmoid(x):
    return 1.0 / (1.0 + jnp.exp(-x))


def _silu(x):
    return x * _sigmoid(x)


def _gate_body(x_ref, gwt_ref, gb_ref, pos0_ref, pos1_ref, w0_ref, w1_ref,
               be_ref, nb_ref):
    x = x_ref[...]
    logits = jnp.dot(x, gwt_ref[...], preferred_element_type=F32)
    st = _sigmoid(jnp.transpose(logits))
    s = st + gb_ref[...]

    erow = lax.broadcasted_iota(I32, (E, T), 0)

    gsc = []
    for g in range(G):
        rows = [s[g * 4 + j:g * 4 + j + 1, :] for j in range(4)]
        best = rows[0] + rows[1]
        for a in range(4):
            for b in range(a + 1, 4):
                if (a, b) != (0, 1):
                    best = jnp.maximum(best, rows[a] + rows[b])
        gsc.append(best)

    keep16 = jnp.zeros((E, T), dtype=jnp.bool_)
    for g in range(G):
        rank = jnp.zeros((1, T), dtype=I32)
        for g2 in range(G):
            if g2 == g:
                continue
            beats = (gsc[g2] > gsc[g]) | ((gsc[g2] == gsc[g]) & (g2 < g))
            rank = rank + beats.astype(I32)
        keep16 = keep16 | (((erow // 4) == g) & (rank < TOPK_G))

    sm = jnp.where(keep16, s, -1e30)

    r = jnp.zeros((E, T), dtype=I32)
    for e2 in range(E):
        row = sm[e2:e2 + 1, :]
        beats = (row > sm) | ((row == sm) & (e2 < erow))
        r = r + beats.astype(I32)
    sel = r < K

    wsel = jnp.where(sel, st, 0.0)
    denom = jnp.sum(wsel, axis=0, keepdims=True) + 1e-6
    cw = wsel / denom * RS

    c = sel.astype(F32)
    sh = 1
    while sh < T:
        c = c + jnp.concatenate(
            [jnp.zeros((E, sh), F32), c[:, :T - sh]], axis=1)
        sh *= 2
    rank_t = c - sel.astype(F32)
    counts = c[:, T - 1:T]

    pc = jnp.ceil(counts / BT) * BT
    p = pc
    sh = 1
    while sh < E:
        p = p + jnp.concatenate([jnp.zeros((sh, 1), F32), p[:E - sh, :]],
                                axis=0)
        sh *= 2
    po = p - pc

    pos16 = po + rank_t

    is0 = sel & (r == 0)
    is1 = sel & (r == 1)
    pos0 = jnp.sum(jnp.where(is0, pos16, 0.0), axis=0, keepdims=True)
    pos1 = jnp.sum(jnp.where(is1, pos16, 0.0), axis=0, keepdims=True)
    w0 = jnp.sum(jnp.where(is0, cw, 0.0), axis=0, keepdims=True)
    w1 = jnp.sum(jnp.where(is1, cw, 0.0), axis=0, keepdims=True)
    pos0_ref[...] = jnp.transpose(pos0).astype(I32)
    pos1_ref[...] = jnp.transpose(pos1).astype(I32)
    w0_ref[...] = jnp.transpose(w0)
    w1_ref[...] = jnp.transpose(w1)

    bi = (lax.broadcasted_iota(I32, (NB, E), 0) * BT).astype(F32)
    cmp = (jnp.broadcast_to(jnp.transpose(po), (NB, E)) <= bi).astype(I32)
    be = jnp.sum(cmp, axis=1, keepdims=True) - 1
    be_ref[...] = jnp.clip(be, 0, E - 1)
    total_pad = jnp.sum(pc, axis=0, keepdims=True)
    nb_ref[...] = (total_pad / BT).astype(I32)


def _gating(xf, gate_weight, gate_bias):
    out_shapes = (
        jax.ShapeDtypeStruct((T, 1), I32),
        jax.ShapeDtypeStruct((T, 1), I32),
        jax.ShapeDtypeStruct((T, 1), F32),
        jax.ShapeDtypeStruct((T, 1), F32),
        jax.ShapeDtypeStruct((NB, 1), I32),
        jax.ShapeDtypeStruct((1, 1), I32),
    )
    return pl.pallas_call(
        _gate_body,
        out_shape=out_shapes,
    )(xf, gate_weight.T, gate_bias.reshape(E, 1))


def _sc_scatter_body(xf_hbm, pos0_hbm, pos1_hbm, xg_hbm, idx0_v, idx1_v,
                     rows_v, sem):
    wid = lax.axis_index("s") * NC + lax.axis_index("c")
    base = wid * TPW
    pltpu.sync_copy(pos0_hbm.at[pl.ds(base, TPW)], idx0_v)
    pltpu.sync_copy(pos1_hbm.at[pl.ds(base, TPW)], idx1_v)
    pltpu.sync_copy(xf_hbm.at[pl.ds(base, TPW)], rows_v)
    cp0 = pltpu.async_copy(rows_v, xg_hbm.at[idx0_v], sem)
    cp1 = pltpu.async_copy(rows_v, xg_hbm.at[idx1_v], sem)
    cp0.wait()
    cp1.wait()


def _sc_scatter(xf, pos0, pos1):
    mesh = plsc.VectorSubcoreMesh(core_axis_name="c", subcore_axis_name="s",
                                  num_cores=NC, num_subcores=NS)
    return pl.kernel(
        _sc_scatter_body,
        out_type=jax.ShapeDtypeStruct((NB * BT, D), F32),
        mesh=mesh,
        scratch_types=[
            pltpu.VMEM((TPW,), I32),
            pltpu.VMEM((TPW,), I32),
            pltpu.VMEM((TPW, D), F32),
            pltpu.SemaphoreType.DMA,
        ],
    )(xf, pos0, pos1)


def _w_copies(w1_ref, w3_ref, w2_ref, e, s, w1s, w3s, w2s, sem1, sem3, sem2):
    return (
        pltpu.make_async_copy(w1_ref.at[e], w1s.at[s], sem1.at[s]),
        pltpu.make_async_copy(w3_ref.at[e], w3s.at[s], sem3.at[s]),
        pltpu.make_async_copy(w2_ref.at[e], w2s.at[s], sem2.at[s]),
    )


def _ffn_body(be_ref, nb_ref, xg_ref, w1_ref, w3_ref, w2_ref, out_ref,
              w1s, w3s, w2s, w1b, w3b, w2b, cnt_ref, sem1, sem3, sem2):
    b = pl.program_id(0)
    prev = be_ref[jnp.maximum(b - 1, 0)]
    changed = jnp.logical_or(b == 0, be_ref[b] != prev)

    @pl.when(b == 0)
    def _():
        cnt_ref[0] = 0
        for cp in _w_copies(w1_ref, w3_ref, w2_ref, be_ref[0], 0,
                            w1s, w3s, w2s, sem1, sem3, sem2):
            cp.start()

    @pl.when(jnp.logical_and(b > 0, changed))
    def _():
        cnt_ref[0] = cnt_ref[0] + 1

    slot = lax.rem(cnt_ref[0], 2)

    @pl.when(changed)
    def _():
        for cp in _w_copies(w1_ref, w3_ref, w2_ref, be_ref[b], slot,
                            w1s, w3s, w2s, sem1, sem3, sem2):
            cp.wait()
        w1b[...] = w1s[slot].astype(BF16)
        w3b[...] = w3s[slot].astype(BF16)
        w2b[...] = w2s[slot].astype(BF16)

    nxt = be_ref[jnp.minimum(b + 1, NB - 1)]
    do_pf = jnp.logical_and(b + 1 < NB, nxt != be_ref[b])

    @pl.when(do_pf)
    def _():
        for cp in _w_copies(w1_ref, w3_ref, w2_ref, nxt, 1 - slot,
                            w1s, w3s, w2s, sem1, sem3, sem2):
            cp.start()

    @pl.when(b < nb_ref[0])
    def _():
        x = xg_ref[...].astype(BF16)
        h1 = lax.dot_general(x, w1b[...], (((1,), (1,)), ((), ())),
                             preferred_element_type=F32)
        h3 = lax.dot_general(x, w3b[...], (((1,), (1,)), ((), ())),
                             preferred_element_type=F32)
        g = (_silu(h1) * h3).astype(BF16)
        out_ref[...] = lax.dot_general(g, w2b[...],
                                       (((1,), (1,)), ((), ())),
                                       preferred_element_type=F32)


def _grouped_ffn(xg, be, nb, W1, W2, W3):
    grid_spec = pltpu.PrefetchScalarGridSpec(
        num_scalar_prefetch=2,
        grid=(NB,),
        in_specs=[
            pl.BlockSpec((BT, D), lambda b, be, nb: (b, 0)),
            pl.BlockSpec(memory_space=pl.ANY),
            pl.BlockSpec(memory_space=pl.ANY),
            pl.BlockSpec(memory_space=pl.ANY),
        ],
        out_specs=pl.BlockSpec((BT, D), lambda b, be, nb: (b, 0)),
        scratch_shapes=[
            pltpu.VMEM((2, I, D), F32),
            pltpu.VMEM((2, I, D), F32),
            pltpu.VMEM((2, D, I), F32),
            pltpu.VMEM((I, D), BF16),
            pltpu.VMEM((I, D), BF16),
            pltpu.VMEM((D, I), BF16),
            pltpu.SMEM((1,), I32),
            pltpu.SemaphoreType.DMA((2,)),
            pltpu.SemaphoreType.DMA((2,)),
            pltpu.SemaphoreType.DMA((2,)),
        ],
    )
    return pl.pallas_call(
        _ffn_body,
        grid_spec=grid_spec,
        out_shape=jax.ShapeDtypeStruct((NB * BT, D), F32),
    )(be, nb, xg, W1, W3, W2)


def _sc_gather_body(yg_hbm, pos0_hbm, pos1_hbm, ya_hbm, yb_hbm, idx0_v,
                    idx1_v, rows0_v, rows1_v, sem0, sem1):
    wid = lax.axis_index("s") * NC + lax.axis_index("c")
    base = wid * TPW
    pltpu.sync_copy(pos0_hbm.at[pl.ds(base, TPW)], idx0_v)
    pltpu.sync_copy(pos1_hbm.at[pl.ds(base, TPW)], idx1_v)
    cp0 = pltpu.async_copy(yg_hbm.at[idx0_v], rows0_v, sem0)
    cp1 = pltpu.async_copy(yg_hbm.at[idx1_v], rows1_v, sem1)
    cp0.wait()
    pltpu.sync_copy(rows0_v, ya_hbm.at[pl.ds(base, TPW)])
    cp1.wait()
    pltpu.sync_copy(rows1_v, yb_hbm.at[pl.ds(base, TPW)])


def _sc_gather(yg, pos0, pos1):
    mesh = plsc.VectorSubcoreMesh(core_axis_name="c", subcore_axis_name="s",
                                  num_cores=NC, num_subcores=NS)
    return pl.kernel(
        _sc_gather_body,
        out_type=(jax.ShapeDtypeStruct((T, D), F32),
                  jax.ShapeDtypeStruct((T, D), F32)),
        mesh=mesh,
        scratch_types=[
            pltpu.VMEM((TPW,), I32),
            pltpu.VMEM((TPW,), I32),
            pltpu.VMEM((TPW, D), F32),
            pltpu.VMEM((TPW, D), F32),
            pltpu.SemaphoreType.DMA,
            pltpu.SemaphoreType.DMA,
        ],
    )(yg, pos0, pos1)


def _shared_body(x_ref, sw1_ref, sw3_ref, sw2_ref, ya_ref, yb_ref, w0_ref,
                 w1_ref, out_ref, sw1b_ref, sw3b_ref, sw2b_ref):
    @pl.when(pl.program_id(0) == 0)
    def _():
        sw1b_ref[...] = sw1_ref[...].astype(BF16)
        sw3b_ref[...] = sw3_ref[...].astype(BF16)
        sw2b_ref[...] = sw2_ref[...].astype(BF16)

    x = x_ref[...].astype(BF16)
    h1 = lax.dot_general(x, sw1b_ref[...], (((1,), (1,)), ((), ())),
                         preferred_element_type=F32)
    h3 = lax.dot_general(x, sw3b_ref[...], (((1,), (1,)), ((), ())),
                         preferred_element_type=F32)
    g = (_silu(h1) * h3).astype(BF16)
    z = lax.dot_general(g, sw2b_ref[...], (((1,), (1,)), ((), ())),
                        preferred_element_type=F32)
    out_ref[...] = (z + w0_ref[...] * ya_ref[...]
                    + w1_ref[...] * yb_ref[...])


def _shared_combine(xf, sw1, sw2, sw3, ya, yb, w0, w1):
    BS = 256
    nblk = T // BS
    return pl.pallas_call(
        _shared_body,
        grid=(nblk,),
        in_specs=[
            pl.BlockSpec((BS, D), lambda b: (b, 0)),
            pl.BlockSpec(sw1.shape, lambda b: (0, 0)),
            pl.BlockSpec(sw3.shape, lambda b: (0, 0)),
            pl.BlockSpec(sw2.shape, lambda b: (0, 0)),
            pl.BlockSpec((BS, D), lambda b: (b, 0)),
            pl.BlockSpec((BS, D), lambda b: (b, 0)),
            pl.BlockSpec((BS, 1), lambda b: (b, 0)),
            pl.BlockSpec((BS, 1), lambda b: (b, 0)),
        ],
        out_specs=pl.BlockSpec((BS, D), lambda b: (b, 0)),
        out_shape=jax.ShapeDtypeStruct((T, D), F32),
        scratch_shapes=[
            pltpu.VMEM(sw1.shape, BF16),
            pltpu.VMEM(sw3.shape, BF16),
            pltpu.VMEM(sw2.shape, BF16),
        ],
    )(xf, sw1, sw3, sw2, ya, yb, w0, w1)


@jax.jit
def kernel(x, gate_weight, gate_bias, W1, W2, W3, sw1, sw2, sw3):
    Bb, Ss, Dd = x.shape
    xf = x.reshape(T, D)

    pos0, pos1, w0, w1, be, nb = _gating(xf, gate_weight, gate_bias)
    pos0v = pos0.reshape(T)
    pos1v = pos1.reshape(T)
    bev = be.reshape(NB)
    nbv = nb.reshape(1)

    xg = _sc_scatter(xf, pos0v, pos1v)
    yg = _grouped_ffn(xg, bev, nbv, W1, W2, W3)
    ya, yb = _sc_gather(yg, pos0v, pos1v)
    out = _shared_combine(xf, sw1, sw2, sw3, ya, yb, w0, w1)
    return out.reshape(Bb, Ss, Dd)


# speedup vs baseline: 1.1225x; 1.0140x over previous
import jax
import jax.numpy as jnp
from jax import lax
from jax.experimental import pallas as pl
from jax.experimental.pallas import tpu as pltpu
from jax.experimental.pallas import tpu_sc as plsc

F32 = jnp.float32
BF16 = jnp.bfloat16
I32 = jnp.int32

T = 2048
D = 768
E = 16
I = 512
G = 4
TOPK_G = 2
K = 2
BT = 256
NB = (T * K) // BT + E
RS = 1.0

NC, NS = 2, 16
NW = NC * NS
TPW = T // NW


def _sigmoid(x):
    return 1.0 / (1.0 + jnp.exp(-x))


def _silu(x):
    return x * _sigmoid(x)


def _gate_body(x_ref, gw_ref, gb_ref, pos0_ref, pos1_ref, w0_ref, w1_ref,
               be_ref, nb_ref):
    x = x_ref[...]
    logits_t = lax.dot_general(gw_ref[...], x, (((1,), (1,)), ((), ())),
                               preferred_element_type=F32)
    st = _sigmoid(logits_t)
    s = st + gb_ref[...]

    erow = lax.broadcasted_iota(I32, (E, T), 0)

    gsc = []
    for g in range(G):
        rows = [s[g * 4 + j:g * 4 + j + 1, :] for j in range(4)]
        best = rows[0] + rows[1]
        for a in range(4):
            for b in range(a + 1, 4):
                if (a, b) != (0, 1):
                    best = jnp.maximum(best, rows[a] + rows[b])
        gsc.append(best)

    keep16 = jnp.zeros((E, T), dtype=jnp.bool_)
    for g in range(G):
        rank = jnp.zeros((1, T), dtype=I32)
        for g2 in range(G):
            if g2 == g:
                continue
            beats = (gsc[g2] > gsc[g]) | ((gsc[g2] == gsc[g]) & (g2 < g))
            rank = rank + beats.astype(I32)
        keep16 = keep16 | (((erow // 4) == g) & (rank < TOPK_G))

    sm = jnp.where(keep16, s, -1e30)

    r = jnp.zeros((E, T), dtype=I32)
    for e2 in range(E):
        row = sm[e2:e2 + 1, :]
        beats = (row > sm) | ((row == sm) & (e2 < erow))
        r = r + beats.astype(I32)
    sel = r < K

    wsel = jnp.where(sel, st, 0.0)
    denom = jnp.sum(wsel, axis=0, keepdims=True) + 1e-6
    cw = wsel / denom * RS

    c = sel.astype(F32)
    sh = 1
    while sh < T:
        c = c + jnp.concatenate(
            [jnp.zeros((E, sh), F32), c[:, :T - sh]], axis=1)
        sh *= 2
    rank_t = c - sel.astype(F32)
    counts = c[:, T - 1:T]

    pc = jnp.ceil(counts / BT) * BT
    p = pc
    sh = 1
    while sh < E:
        p = p + jnp.concatenate([jnp.zeros((sh, 1), F32), p[:E - sh, :]],
                                axis=0)
        sh *= 2
    po = p - pc

    pos16 = po + rank_t

    is0 = sel & (r == 0)
    is1 = sel & (r == 1)
    pos0 = jnp.sum(jnp.where(is0, pos16, 0.0), axis=0, keepdims=True)
    pos1 = jnp.sum(jnp.where(is1, pos16, 0.0), axis=0, keepdims=True)
    w0 = jnp.sum(jnp.where(is0, cw, 0.0), axis=0, keepdims=True)
    w1 = jnp.sum(jnp.where(is1, cw, 0.0), axis=0, keepdims=True)
    pos0_ref[...] = jnp.transpose(pos0).astype(I32)
    pos1_ref[...] = jnp.transpose(pos1).astype(I32)
    w0_ref[...] = jnp.transpose(w0)
    w1_ref[...] = jnp.transpose(w1)

    bi = (lax.broadcasted_iota(I32, (NB, E), 0) * BT).astype(F32)
    cmp = (jnp.broadcast_to(jnp.transpose(po), (NB, E)) <= bi).astype(I32)
    be = jnp.sum(cmp, axis=1, keepdims=True) - 1
    be_ref[...] = jnp.clip(be, 0, E - 1)
    total_pad = jnp.sum(pc, axis=0, keepdims=True)
    nb_ref[...] = (total_pad / BT).astype(I32)


def _gating(xf, gate_weight, gate_bias):
    out_shapes = (
        jax.ShapeDtypeStruct((T, 1), I32),
        jax.ShapeDtypeStruct((T, 1), I32),
        jax.ShapeDtypeStruct((T, 1), F32),
        jax.ShapeDtypeStruct((T, 1), F32),
        jax.ShapeDtypeStruct((NB, 1), I32),
        jax.ShapeDtypeStruct((1, 1), I32),
    )
    return pl.pallas_call(
        _gate_body,
        out_shape=out_shapes,
    )(xf, gate_weight, gate_bias.reshape(E, 1))


def _sc_scatter_body(xf_hbm, pos0_hbm, pos1_hbm, xg_hbm, idx0_v, idx1_v,
                     rows_v, sem):
    wid = lax.axis_index("s") * NC + lax.axis_index("c")
    base = wid * TPW
    pltpu.sync_copy(pos0_hbm.at[pl.ds(base, TPW)], idx0_v)
    pltpu.sync_copy(pos1_hbm.at[pl.ds(base, TPW)], idx1_v)
    pltpu.sync_copy(xf_hbm.at[pl.ds(base, TPW)], rows_v)
    cp0 = pltpu.async_copy(rows_v, xg_hbm.at[idx0_v], sem)
    cp1 = pltpu.async_copy(rows_v, xg_hbm.at[idx1_v], sem)
    cp0.wait()
    cp1.wait()


def _sc_scatter(xf, pos0, pos1):
    mesh = plsc.VectorSubcoreMesh(core_axis_name="c", subcore_axis_name="s",
                                  num_cores=NC, num_subcores=NS)
    return pl.kernel(
        _sc_scatter_body,
        out_type=jax.ShapeDtypeStruct((NB * BT, D), F32),
        mesh=mesh,
        scratch_types=[
            pltpu.VMEM((TPW,), I32),
            pltpu.VMEM((TPW,), I32),
            pltpu.VMEM((TPW, D), F32),
            pltpu.SemaphoreType.DMA,
        ],
    )(xf, pos0, pos1)


def _w_copies(w1_ref, w3_ref, w2_ref, e, s, w1s, w3s, w2s, sem1, sem3, sem2):
    return (
        pltpu.make_async_copy(w1_ref.at[e], w1s.at[s], sem1.at[s]),
        pltpu.make_async_copy(w3_ref.at[e], w3s.at[s], sem3.at[s]),
        pltpu.make_async_copy(w2_ref.at[e], w2s.at[s], sem2.at[s]),
    )


def _ffn_body(be_ref, nb_ref, xg_ref, w1_ref, w3_ref, w2_ref, out_ref,
              w1s, w3s, w2s, w1b, w3b, w2b, cnt_ref, sem1, sem3, sem2):
    b = pl.program_id(0)
    prev = be_ref[jnp.maximum(b - 1, 0)]
    changed = jnp.logical_or(b == 0, be_ref[b] != prev)

    @pl.when(b == 0)
    def _():
        cnt_ref[0] = 0
        for cp in _w_copies(w1_ref, w3_ref, w2_ref, be_ref[0], 0,
                            w1s, w3s, w2s, sem1, sem3, sem2):
            cp.start()

    @pl.when(jnp.logical_and(b > 0, changed))
    def _():
        cnt_ref[0] = cnt_ref[0] + 1

    slot = lax.rem(cnt_ref[0], 2)

    @pl.when(changed)
    def _():
        for cp in _w_copies(w1_ref, w3_ref, w2_ref, be_ref[b], slot,
                            w1s, w3s, w2s, sem1, sem3, sem2):
            cp.wait()
        w1b[...] = w1s[slot].astype(BF16)
        w3b[...] = w3s[slot].astype(BF16)
        w2b[...] = w2s[slot].astype(BF16)

    nxt = be_ref[jnp.minimum(b + 1, NB - 1)]
    do_pf = jnp.logical_and(b + 1 < NB, nxt != be_ref[b])

    @pl.when(do_pf)
    def _():
        for cp in _w_copies(w1_ref, w3_ref, w2_ref, nxt, 1 - slot,
                            w1s, w3s, w2s, sem1, sem3, sem2):
            cp.start()

    @pl.when(b < nb_ref[0])
    def _():
        x = xg_ref[...].astype(BF16)
        h1 = lax.dot_general(x, w1b[...], (((1,), (1,)), ((), ())),
                             preferred_element_type=F32)
        h3 = lax.dot_general(x, w3b[...], (((1,), (1,)), ((), ())),
                             preferred_element_type=F32)
        g = (_silu(h1) * h3).astype(BF16)
        out_ref[...] = lax.dot_general(g, w2b[...],
                                       (((1,), (1,)), ((), ())),
                                       preferred_element_type=F32)


def _grouped_ffn(xg, be, nb, W1, W2, W3):
    grid_spec = pltpu.PrefetchScalarGridSpec(
        num_scalar_prefetch=2,
        grid=(NB,),
        in_specs=[
            pl.BlockSpec((BT, D), lambda b, be, nb: (b, 0)),
            pl.BlockSpec(memory_space=pl.ANY),
            pl.BlockSpec(memory_space=pl.ANY),
            pl.BlockSpec(memory_space=pl.ANY),
        ],
        out_specs=pl.BlockSpec((BT, D), lambda b, be, nb: (b, 0)),
        scratch_shapes=[
            pltpu.VMEM((2, I, D), F32),
            pltpu.VMEM((2, I, D), F32),
            pltpu.VMEM((2, D, I), F32),
            pltpu.VMEM((I, D), BF16),
            pltpu.VMEM((I, D), BF16),
            pltpu.VMEM((D, I), BF16),
            pltpu.SMEM((1,), I32),
            pltpu.SemaphoreType.DMA((2,)),
            pltpu.SemaphoreType.DMA((2,)),
            pltpu.SemaphoreType.DMA((2,)),
        ],
    )
    return pl.pallas_call(
        _ffn_body,
        grid_spec=grid_spec,
        out_shape=jax.ShapeDtypeStruct((NB * BT, D), F32),
    )(be, nb, xg, W1, W3, W2)


def _sc_gather_body(yg_hbm, pos0_hbm, pos1_hbm, ya_hbm, yb_hbm, idx0_v,
                    idx1_v, rows0_v, rows1_v, sem0, sem1):
    wid = lax.axis_index("s") * NC + lax.axis_index("c")
    base = wid * TPW
    pltpu.sync_copy(pos0_hbm.at[pl.ds(base, TPW)], idx0_v)
    pltpu.sync_copy(pos1_hbm.at[pl.ds(base, TPW)], idx1_v)
    cp0 = pltpu.async_copy(yg_hbm.at[idx0_v], rows0_v, sem0)
    cp1 = pltpu.async_copy(yg_hbm.at[idx1_v], rows1_v, sem1)
    cp0.wait()
    pltpu.sync_copy(rows0_v, ya_hbm.at[pl.ds(base, TPW)])
    cp1.wait()
    pltpu.sync_copy(rows1_v, yb_hbm.at[pl.ds(base, TPW)])


def _sc_gather(yg, pos0, pos1):
    mesh = plsc.VectorSubcoreMesh(core_axis_name="c", subcore_axis_name="s",
                                  num_cores=NC, num_subcores=NS)
    return pl.kernel(
        _sc_gather_body,
        out_type=(jax.ShapeDtypeStruct((T, D), F32),
                  jax.ShapeDtypeStruct((T, D), F32)),
        mesh=mesh,
        scratch_types=[
            pltpu.VMEM((TPW,), I32),
            pltpu.VMEM((TPW,), I32),
            pltpu.VMEM((TPW, D), F32),
            pltpu.VMEM((TPW, D), F32),
            pltpu.SemaphoreType.DMA,
            pltpu.SemaphoreType.DMA,
        ],
    )(yg, pos0, pos1)


def _shared_body(x_ref, sw1_ref, sw3_ref, sw2_ref, ya_ref, yb_ref, w0_ref,
                 w1_ref, out_ref, sw1b_ref, sw3b_ref, sw2b_ref):
    @pl.when(pl.program_id(0) == 0)
    def _():
        sw1b_ref[...] = sw1_ref[...].astype(BF16)
        sw3b_ref[...] = sw3_ref[...].astype(BF16)
        sw2b_ref[...] = sw2_ref[...].astype(BF16)

    x = x_ref[...].astype(BF16)
    h1 = lax.dot_general(x, sw1b_ref[...], (((1,), (1,)), ((), ())),
                         preferred_element_type=F32)
    h3 = lax.dot_general(x, sw3b_ref[...], (((1,), (1,)), ((), ())),
                         preferred_element_type=F32)
    g = (_silu(h1) * h3).astype(BF16)
    z = lax.dot_general(g, sw2b_ref[...], (((1,), (1,)), ((), ())),
                        preferred_element_type=F32)
    out_ref[...] = (z + w0_ref[...] * ya_ref[...]
                    + w1_ref[...] * yb_ref[...])


def _shared_combine(xf, sw1, sw2, sw3, ya, yb, w0, w1):
    BS = 256
    nblk = T // BS
    return pl.pallas_call(
        _shared_body,
        grid=(nblk,),
        in_specs=[
            pl.BlockSpec((BS, D), lambda b: (b, 0)),
            pl.BlockSpec(sw1.shape, lambda b: (0, 0)),
            pl.BlockSpec(sw3.shape, lambda b: (0, 0)),
            pl.BlockSpec(sw2.shape, lambda b: (0, 0)),
            pl.BlockSpec((BS, D), lambda b: (b, 0)),
            pl.BlockSpec((BS, D), lambda b: (b, 0)),
            pl.BlockSpec((BS, 1), lambda b: (b, 0)),
            pl.BlockSpec((BS, 1), lambda b: (b, 0)),
        ],
        out_specs=pl.BlockSpec((BS, D), lambda b: (b, 0)),
        out_shape=jax.ShapeDtypeStruct((T, D), F32),
        scratch_shapes=[
            pltpu.VMEM(sw1.shape, BF16),
            pltpu.VMEM(sw3.shape, BF16),
            pltpu.VMEM(sw2.shape, BF16),
        ],
    )(xf, sw1, sw3, sw2, ya, yb, w0, w1)


@jax.jit
def kernel(x, gate_weight, gate_bias, W1, W2, W3, sw1, sw2, sw3):
    Bb, Ss, Dd = x.shape
    xf = x.reshape(T, D)

    pos0, pos1, w0, w1, be, nb = _gating(xf, gate_weight, gate_bias)
    pos0v = pos0.reshape(T)
    pos1v = pos1.reshape(T)
    bev = be.reshape(NB)
    nbv = nb.reshape(1)

    xg = _sc_scatter(xf, pos0v, pos1v)
    yg = _grouped_ffn(xg, bev, nbv, W1, W2, W3)
    ya, yb = _sc_gather(yg, pos0v, pos1v)
    out = _shared_combine(xf, sw1, sw2, sw3, ya, yb, w0, w1)
    return out.reshape(Bb, Ss, Dd)
